# baseline (speedup 1.0000x reference)
import jax
import jax.numpy as jnp
from jax.experimental import pallas as pl
from jax.experimental.pallas import tpu as pltpu

NUM_EXPERTS = 16
HIDDEN = 768
CAP_FACTORS = [1.5, 1.5, 1.5, 1.5, 1.0, 1.0, 1.0, 1.0, 1.0, 1.0, 1.0, 1.0, 0.5, 0.5, 0.5, 0.5]
BASE_CAP = 128
MAX_CAP = int(max(CAP_FACTORS) * BASE_CAP)


def _fused_kernel(x_ref, w_ref, b_ref, caps_ref,
                  disp_ref, comb_ref, zsum_ref,
                  work_ref, valsT_ref, idxT_ref):
    i = pl.program_id(0)
    R, T = work_ref.shape
    C = MAX_CAP
    G = x_ref.shape[0]
    E = R // G

    @pl.when(i == 0)
    def _select():
        w = w_ref[...]
        zsum = jnp.zeros((1, 1), jnp.float32)
        for g in range(G):
            xg = x_ref[g]
            logits = jax.lax.dot_general(
                w, xg, (((1,), (1,)), ((), ())),
                preferred_element_type=jnp.float32)
            logits = logits + b_ref[...].T
            m = jnp.max(logits, axis=0, keepdims=True)
            e = jnp.exp(logits - m)
            s = jnp.sum(e, axis=0, keepdims=True)
            p = e / s
            work_ref[4 * g:4 * g + 4, :] = p[0:4]
            work_ref[8 + 8 * g:8 + 8 * g + 8, :] = p[4:12]
            work_ref[24 + 4 * g:24 + 4 * g + 4, :] = p[12:16]
            logz = m + jnp.log(s)
            zsum = zsum + jnp.sum(logz * logz).reshape(1, 1)
        zsum_ref[...] = zsum

        UNROLL = 8

        def make_body(nrows):
            iota_t = jax.lax.broadcasted_iota(jnp.int32, (nrows, T), 1)
            iota_c = jax.lax.broadcasted_iota(jnp.int32, (nrows, C), 1)

            def body(c, carry):
                vals, idxs, cur = carry
                for u in range(UNROLL):
                    mx = jnp.max(cur, axis=1, keepdims=True)
                    idx = jnp.argmax(cur, axis=1)[:, None]
                    sel = iota_c == UNROLL * c + u
                    vals = jnp.where(sel, mx, vals)
                    idxs = jnp.where(sel, idx, idxs)
                    cur = jnp.where(iota_t == idx, -jnp.inf, cur)
                return (vals, idxs, cur)

            return body

        vals0 = jnp.zeros((R, C), jnp.float32)
        idx0 = jnp.zeros((R, C), jnp.int32)
        n1 = (BASE_CAP // 2) // UNROLL
        n2 = BASE_CAP // UNROLL
        n3 = C // UNROLL
        vals1, idxs1, cur1 = jax.lax.fori_loop(
            0, n1, make_body(R), (vals0, idx0, work_ref[...]))
        vals2, idxs2, cur2 = jax.lax.fori_loop(
            n1, n2, make_body(24), (vals1[0:24], idxs1[0:24], cur1[0:24]))
        vals3, idxs3, _ = jax.lax.fori_loop(
            n2, n3, make_body(8), (vals2[0:8], idxs2[0:8], cur2[0:8]))
        vals = jnp.concatenate([vals3, vals2[8:24], vals1[24:R]], axis=0)
        idxs = jnp.concatenate([idxs3, idxs2[8:24], idxs1[24:R]], axis=0)

        iota_c = jax.lax.broadcasted_iota(jnp.int32, (R, C), 1)
        caps = caps_ref[:, 0:1]
        live = iota_c < caps
        valsT_ref[...] = jnp.where(live, vals, 0.0).T
        idxT_ref[...] = jnp.where(live, idxs, -1).T

    @pl.when(i > 0)
    def _materialize():
        lane_r = jax.lax.broadcasted_iota(jnp.int32, (C, R), 1)
        tid = jax.lax.broadcasted_iota(jnp.int32, (1, T), 1)
        for j in range(4):
            r = 4 * (i - 1) + j
            g = r // E
            e = r % E
            prow = jnp.where(e < 4, 4 * g + e,
                             jnp.where(e < 12, 4 + 8 * g + e,
                                       12 + 4 * g + e))
            pick = lane_r == prow
            val_col = jnp.sum(jnp.where(pick, valsT_ref[...], 0.0),
                              axis=1, keepdims=True)
            idx_col = jnp.sum(jnp.where(pick, idxT_ref[...], 0),
                              axis=1, keepdims=True)
            hit = idx_col == tid
            comb_ref[0, j] = jnp.where(hit, val_col, 0.0)
            disp_ref[0, j] = hit


def kernel(token_inputs, W, b, expert_capacity):
    x = token_inputs.astype(jnp.float32)
    G, T, H = x.shape
    E = NUM_EXPERTS
    C = MAX_CAP
    R = G * E

    factors_perm = jnp.asarray([1.5] * 8 + [1.0] * 16 + [0.5] * 8,
                               dtype=jnp.float32)
    caps_perm = jnp.floor(factors_perm * expert_capacity).astype(jnp.int32)
    caps_rows = jnp.broadcast_to(caps_perm[:, None], (R, 128))

    def _ge(i):
        s = jnp.maximum(i - 1, 0)
        return (s // (E // 4), s % (E // 4), 0, 0)

    disp_ect, comb_ect, zsum = pl.pallas_call(
        _fused_kernel,
        grid=(1 + G * E // 4,),
        in_specs=[
            pl.BlockSpec((G, T, H), lambda i: (0, 0, 0)),
            pl.BlockSpec((E, H), lambda i: (0, 0)),
            pl.BlockSpec((1, E), lambda i: (0, 0)),
            pl.BlockSpec((R, 128), lambda i: (0, 0)),
        ],
        out_specs=[
            pl.BlockSpec((1, 4, C, T), _ge),
            pl.BlockSpec((1, 4, C, T), _ge),
            pl.BlockSpec((1, 1), lambda i: (0, 0)),
        ],
        out_shape=[
            jax.ShapeDtypeStruct((G, E, C, T), jnp.bool_),
            jax.ShapeDtypeStruct((G, E, C, T), jnp.float32),
            jax.ShapeDtypeStruct((1, 1), jnp.float32),
        ],
        scratch_shapes=[
            pltpu.VMEM((R, T), jnp.float32),
            pltpu.VMEM((C, R), jnp.float32),
            pltpu.VMEM((C, R), jnp.int32),
        ],
    )(x, W, b.reshape(1, E), caps_rows)

    router_z_loss = (zsum[0, 0] / (G * T)).astype(jnp.float32)
    auxiliary_loss = jnp.zeros((), dtype=jnp.float32)

    dispatch_mask = jnp.transpose(disp_ect, (0, 3, 1, 2))
    combine_array = jnp.transpose(comb_ect, (0, 3, 1, 2))
    return (dispatch_mask, combine_array, auxiliary_loss, router_z_loss)


# speedup vs baseline: 1.0166x; 1.0166x over previous
import jax
import jax.numpy as jnp
from jax.experimental import pallas as pl
from jax.experimental.pallas import tpu as pltpu

NUM_EXPERTS = 16
HIDDEN = 768
CAP_FACTORS = [1.5, 1.5, 1.5, 1.5, 1.0, 1.0, 1.0, 1.0, 1.0, 1.0, 1.0, 1.0, 0.5, 0.5, 0.5, 0.5]
BASE_CAP = 128
MAX_CAP = int(max(CAP_FACTORS) * BASE_CAP)


def _fused_kernel(x_ref, w_ref, b_ref, caps_ref,
                  disp_ref, comb_ref, zsum_ref,
                  work_ref, valsT_ref, idxT_ref):
    i = pl.program_id(0)
    R, T = work_ref.shape
    C = MAX_CAP
    G = x_ref.shape[0]
    E = R // G

    @pl.when(i == 0)
    def _select():
        w = w_ref[...]
        zsum = jnp.zeros((1, 1), jnp.float32)
        for g in range(G):
            xg = x_ref[g]
            logits = jax.lax.dot_general(
                w, xg, (((1,), (1,)), ((), ())),
                preferred_element_type=jnp.float32)
            logits = logits + b_ref[...].T
            m = jnp.max(logits, axis=0, keepdims=True)
            e = jnp.exp(logits - m)
            s = jnp.sum(e, axis=0, keepdims=True)
            p = e / s
            work_ref[4 * g:4 * g + 4, :] = p[0:4]
            work_ref[8 + 8 * g:8 + 8 * g + 8, :] = p[4:12]
            work_ref[24 + 4 * g:24 + 4 * g + 4, :] = p[12:16]
            logz = m + jnp.log(s)
            zsum = zsum + jnp.sum(logz * logz).reshape(1, 1)
        zsum_ref[...] = zsum

        UNROLL = 8

        def make_body(nrows):
            iota_t = jax.lax.broadcasted_iota(jnp.int32, (nrows, T), 1)
            iota_c = jax.lax.broadcasted_iota(jnp.int32, (nrows, C), 1)

            def body(c, carry):
                vals, idxs, cur = carry
                for u in range(UNROLL):
                    mx = jnp.max(cur, axis=1, keepdims=True)
                    idx = jnp.argmax(cur, axis=1)[:, None]
                    sel = iota_c == UNROLL * c + u
                    vals = jnp.where(sel, mx, vals)
                    idxs = jnp.where(sel, idx, idxs)
                    cur = jnp.where(iota_t == idx, -jnp.inf, cur)
                return (vals, idxs, cur)

            return body

        vals0 = jnp.zeros((R, C), jnp.float32)
        idx0 = jnp.zeros((R, C), jnp.int32)
        n1 = (BASE_CAP // 2) // UNROLL
        n2 = BASE_CAP // UNROLL
        n3 = C // UNROLL
        vals1, idxs1, cur1 = jax.lax.fori_loop(
            0, n1, make_body(R), (vals0, idx0, work_ref[...]))
        vals2, idxs2, cur2 = jax.lax.fori_loop(
            n1, n2, make_body(24), (vals1[0:24], idxs1[0:24], cur1[0:24]))
        vals3, idxs3, _ = jax.lax.fori_loop(
            n2, n3, make_body(8), (vals2[0:8], idxs2[0:8], cur2[0:8]))
        vals = jnp.concatenate([vals3, vals2[8:24], vals1[24:R]], axis=0)
        idxs = jnp.concatenate([idxs3, idxs2[8:24], idxs1[24:R]], axis=0)

        iota_c = jax.lax.broadcasted_iota(jnp.int32, (R, C), 1)
        caps = caps_ref[:, 0:1]
        live = iota_c < caps
        valsT_ref[...] = jnp.where(live, vals, 0.0).T
        idxT_ref[...] = jnp.where(live, idxs, -1).T

    @pl.when(i > 0)
    def _materialize():
        lane_r = jax.lax.broadcasted_iota(jnp.int32, (C, R), 1)
        tid = jax.lax.broadcasted_iota(jnp.int32, (1, T), 1)
        for j in range(2):
            r = 2 * (i - 1) + j
            g = r // E
            e = r % E
            prow = jnp.where(e < 4, 4 * g + e,
                             jnp.where(e < 12, 4 + 8 * g + e,
                                       12 + 4 * g + e))
            pick = lane_r == prow
            val_col = jnp.sum(jnp.where(pick, valsT_ref[...], 0.0),
                              axis=1, keepdims=True)
            idx_col = jnp.sum(jnp.where(pick, idxT_ref[...], 0),
                              axis=1, keepdims=True)
            hit = idx_col == tid
            comb_ref[0, j] = jnp.where(hit, val_col, 0.0)
            disp_ref[0, j] = hit


def kernel(token_inputs, W, b, expert_capacity):
    x = token_inputs.astype(jnp.float32)
    G, T, H = x.shape
    E = NUM_EXPERTS
    C = MAX_CAP
    R = G * E

    factors_perm = jnp.asarray([1.5] * 8 + [1.0] * 16 + [0.5] * 8,
                               dtype=jnp.float32)
    caps_perm = jnp.floor(factors_perm * expert_capacity).astype(jnp.int32)
    caps_rows = jnp.broadcast_to(caps_perm[:, None], (R, 128))

    def _ge(i):
        s = jnp.maximum(i - 1, 0)
        return (s // (E // 2), s % (E // 2), 0, 0)

    disp_ect, comb_ect, zsum = pl.pallas_call(
        _fused_kernel,
        grid=(1 + G * E // 2,),
        in_specs=[
            pl.BlockSpec((G, T, H), lambda i: (0, 0, 0)),
            pl.BlockSpec((E, H), lambda i: (0, 0)),
            pl.BlockSpec((1, E), lambda i: (0, 0)),
            pl.BlockSpec((R, 128), lambda i: (0, 0)),
        ],
        out_specs=[
            pl.BlockSpec((1, 2, C, T), _ge),
            pl.BlockSpec((1, 2, C, T), _ge),
            pl.BlockSpec((1, 1), lambda i: (0, 0)),
        ],
        out_shape=[
            jax.ShapeDtypeStruct((G, E, C, T), jnp.bool_),
            jax.ShapeDtypeStruct((G, E, C, T), jnp.float32),
            jax.ShapeDtypeStruct((1, 1), jnp.float32),
        ],
        scratch_shapes=[
            pltpu.VMEM((R, T), jnp.float32),
            pltpu.VMEM((C, R), jnp.float32),
            pltpu.VMEM((C, R), jnp.int32),
        ],
    )(x, W, b.reshape(1, E), caps_rows)

    router_z_loss = (zsum[0, 0] / (G * T)).astype(jnp.float32)
    auxiliary_loss = jnp.zeros((), dtype=jnp.float32)

    dispatch_mask = jnp.transpose(disp_ect, (0, 3, 1, 2))
    combine_array = jnp.transpose(comb_ect, (0, 3, 1, 2))
    return (dispatch_mask, combine_array, auxiliary_loss, router_z_loss)
